# Initial kernel scaffold; baseline (speedup 1.0000x reference)
import jax
import jax.numpy as jnp
from jax.experimental import pallas as pl


def kernel(x, edge_index, W_pool, b_pool, W_self, W_neigh, b_sage, W_fc, b_fc, gamma, beta):
    raise NotImplementedError("write your pallas kernel here")


# speedup vs baseline: 1.6444x; 1.6444x over previous
import dataclasses
import functools

import jax
import jax.numpy as jnp
from jax import lax
from jax.experimental import pallas as pl
from jax.experimental.pallas import tpu as pltpu
from jax.experimental.pallas import tpu_sc as plsc

N = 10000
E = 320000
D = 128

NPAD = 10240
NW = 32
RPW = NPAD // NW
TRASH = RPW
CHUNK = 8000
NCHUNK = E // CHUNK
GB = 128
BN_SCALE = float(1.0 / (1.0 + 1e-5) ** 0.5)


def _dense_pre_body(x_ref, wp_ref, bp_ref, ws_ref, m_ref, hws_ref):
    h = jnp.log(x_ref[...] + 1.0)
    m_ref[...] = jax.nn.relu(
        jnp.dot(h, wp_ref[...], preferred_element_type=jnp.float32) + bp_ref[...]
    )
    hws_ref[...] = jnp.dot(h, ws_ref[...], preferred_element_type=jnp.float32)


def _dense_pre(xp, W_pool, b_pool2d, W_self):
    blk = NPAD // 8
    return pl.pallas_call(
        _dense_pre_body,
        grid=(8,),
        in_specs=[
            pl.BlockSpec((blk, D), lambda i: (i, 0)),
            pl.BlockSpec((D, D), lambda i: (0, 0)),
            pl.BlockSpec((1, D), lambda i: (0, 0)),
            pl.BlockSpec((D, D), lambda i: (0, 0)),
        ],
        out_specs=[
            pl.BlockSpec((blk, D), lambda i: (i, 0)),
            pl.BlockSpec((blk, D), lambda i: (i, 0)),
        ],
        out_shape=[
            jax.ShapeDtypeStruct((NPAD, D), jnp.float32),
            jax.ShapeDtypeStruct((NPAD, D), jnp.float32),
        ],
    )(xp, W_pool, b_pool2d, W_self)


def _seg_max_body(src_hbm, dst_hbm, m_hbm, agg_hbm,
                  srcv, dstv, slist, dlist, grows, acc):
    wid = lax.axis_index("s") * 2 + lax.axis_index("c")
    lo = wid * RPW
    lanes = lax.broadcasted_iota(jnp.int32, (16,), 0)

    @pl.loop(0, RPW + 1)
    def _(r):
        @pl.loop(0, D, step=16)
        def _(k):
            acc[r, pl.ds(k, 16)] = jnp.zeros((16,), jnp.float32)

    @pl.loop(0, NCHUNK)
    def _(ci):
        off = ci * CHUNK
        pltpu.sync_copy(src_hbm.at[pl.ds(off, CHUNK)], srcv)
        pltpu.sync_copy(dst_hbm.at[pl.ds(off, CHUNK)], dstv)

        def scan_body(i, ptr):
            d = dstv[pl.ds(i * 16, 16)]
            s = srcv[pl.ds(i * 16, 16)]
            dl = d - lo
            mask = (dl >= 0) & (dl < RPW)
            plsc.store_compressed(slist.at[pl.ds(ptr, 16)], s, mask=mask)
            plsc.store_compressed(dlist.at[pl.ds(ptr, 16)], dl, mask=mask)
            return ptr + jnp.sum(mask.astype(jnp.int32))

        ptr = lax.fori_loop(0, CHUNK // 16, scan_body, jnp.int32(0))

        @pl.loop(0, GB, step=16)
        def _(t):
            slist[pl.ds(ptr + t, 16)] = lanes
            dlist[pl.ds(ptr + t, 16)] = jnp.full((16,), TRASH, jnp.int32)

        nb = (ptr + (GB - 1)) // GB

        def batch_body(b, carry):
            pltpu.sync_copy(m_hbm.at[slist.at[pl.ds(b * GB, GB)]], grows)

            def edge_body(e, c2):
                dl = dlist[pl.ds(b * GB + e, 16)][0]
                for k in range(D // 16):
                    cur = acc[dl, pl.ds(k * 16, 16)]
                    acc[dl, pl.ds(k * 16, 16)] = jnp.maximum(
                        cur, grows[e, pl.ds(k * 16, 16)])
                return c2

            return lax.fori_loop(0, GB, edge_body, carry)

        lax.fori_loop(0, nb, batch_body, jnp.int32(0))

    pltpu.sync_copy(acc.at[pl.ds(0, RPW)], agg_hbm.at[pl.ds(lo, RPW)])


def _seg_max(src, dst, m):
    mesh = plsc.VectorSubcoreMesh(core_axis_name="c", subcore_axis_name="s")
    cp = pltpu.CompilerParams()
    if "needs_layout_passes" in pltpu.CompilerParams.__dataclass_fields__:
        cp = dataclasses.replace(cp, needs_layout_passes=False)
    f = pl.kernel(
        _seg_max_body,
        out_type=jax.ShapeDtypeStruct((NPAD, D), jnp.float32),
        mesh=mesh,
        compiler_params=cp,
        scratch_types=[
            pltpu.VMEM((CHUNK,), jnp.int32),
            pltpu.VMEM((CHUNK,), jnp.int32),
            pltpu.VMEM((CHUNK + GB + 16,), jnp.int32),
            pltpu.VMEM((CHUNK + GB + 16,), jnp.int32),
            pltpu.VMEM((GB, D), jnp.float32),
            pltpu.VMEM((RPW + 1, D), jnp.float32),
        ],
    )
    return f(src, dst, m)


def _dense_post_body(hws_ref, agg_ref, wn_ref, bs_ref, wf_ref, bf_ref,
                     g_ref, be_ref, z_ref):
    rst = jax.nn.relu(
        hws_ref[...]
        + jnp.dot(agg_ref[...], wn_ref[...], preferred_element_type=jnp.float32)
        + bs_ref[...]
    )
    nrm = jnp.maximum(
        jnp.sqrt(jnp.sum(rst * rst, axis=1, keepdims=True)), 1e-12)
    rst = rst / nrm
    z = jnp.dot(rst, wf_ref[...], preferred_element_type=jnp.float32) + bf_ref[...]
    z = z * (BN_SCALE * g_ref[...]) + be_ref[...]
    z_ref[...] = jax.nn.relu(z)


def _dense_post(hws, agg, W_neigh, b_sage2d, W_fc, b_fc2d, gamma2d, beta2d):
    blk = NPAD // 8
    return pl.pallas_call(
        _dense_post_body,
        grid=(8,),
        in_specs=[
            pl.BlockSpec((blk, D), lambda i: (i, 0)),
            pl.BlockSpec((blk, D), lambda i: (i, 0)),
            pl.BlockSpec((D, D), lambda i: (0, 0)),
            pl.BlockSpec((1, D), lambda i: (0, 0)),
            pl.BlockSpec((D, D), lambda i: (0, 0)),
            pl.BlockSpec((1, D), lambda i: (0, 0)),
            pl.BlockSpec((1, D), lambda i: (0, 0)),
            pl.BlockSpec((1, D), lambda i: (0, 0)),
        ],
        out_specs=pl.BlockSpec((blk, D), lambda i: (i, 0)),
        out_shape=jax.ShapeDtypeStruct((NPAD, D), jnp.float32),
    )(hws, agg, W_neigh, b_sage2d, W_fc, b_fc2d, gamma2d, beta2d)


def kernel(x, edge_index, W_pool, b_pool, W_self, W_neigh, b_sage,
           W_fc, b_fc, gamma, beta):
    xp = jnp.zeros((NPAD, D), jnp.float32).at[:N].set(x)
    m, hws = _dense_pre(xp, W_pool, b_pool.reshape(1, D), W_self)
    agg = _seg_max(edge_index[0], edge_index[1], m)
    z = _dense_post(hws, agg, W_neigh, b_sage.reshape(1, D), W_fc,
                    b_fc.reshape(1, D), gamma.reshape(1, D), beta.reshape(1, D))
    return z[:N]
